# baseline (speedup 1.0000x reference)
import jax
import jax.numpy as jnp
from jax import lax
from jax.experimental import pallas as pl
from jax.experimental.pallas import tpu as pltpu

_IMAGE_W = 1024
_IMAGE_H = 1024
_MIN_SCORE = 0.05
_NMS_THR = 0.5
_MAX_DET = 100
_LANES = 128
_NEG_INF = float("-inf")


def _decoder_body(cls_ref, reg_ref, anc_ref,
                  s_out_ref, c_out_ref, b_out_ref,
                  w_ref, a_ref, p_ref):
    B, C, R, L = cls_ref.shape

    def class_step(c, carry):
        m, idx = carry
        v = cls_ref[:, c]
        better = v > m
        return jnp.where(better, v, m), jnp.where(better, c, idx)

    m0 = cls_ref[:, 0]
    idx0 = jnp.zeros((B, R, L), jnp.int32)
    scores, classes = lax.fori_loop(1, C, class_step, (m0, idx0))

    reg = reg_ref[...]
    anc = anc_ref[...]
    ax1, ay1, ax2, ay2 = anc[:, 0], anc[:, 1], anc[:, 2], anc[:, 3]
    aw = ax2 - ax1
    ah = ay2 - ay1
    acx = ax1 + 0.5 * aw
    acy = ay1 + 0.5 * ah
    tx = reg[:, 0] * 0.1
    ty = reg[:, 1] * 0.1
    tw = reg[:, 2] * 0.2
    th = reg[:, 3] * 0.2
    w = jnp.exp(tw) * aw
    h = jnp.exp(th) * ah
    cx = tx * aw + acx
    cy = ty * ah + acy
    xi1 = jnp.maximum((cx - 0.5 * w).astype(jnp.int32), 0)
    yi1 = jnp.maximum((cy - 0.5 * h).astype(jnp.int32), 0)
    xi2 = jnp.minimum((cx + 0.5 * w).astype(jnp.int32), _IMAGE_W - 1)
    yi2 = jnp.minimum((cy + 0.5 * h).astype(jnp.int32), _IMAGE_H - 1)

    w_ref[...] = jnp.where(scores > _MIN_SCORE, scores, _NEG_INF)
    a_ref[...] = (jnp.clip(xi1, 0, 2047)
                  | ((jnp.clip(xi2, -2048, 2047) + 2048) << 11)
                  | (classes << 23))
    p_ref[...] = (jnp.clip(yi1, 0, 2047)
                  | ((jnp.clip(yi2, -2048, 2047) + 2048) << 11))

    rowmax = jnp.max(w_ref[...], axis=2)
    riota = lax.broadcasted_iota(jnp.int32, (B, R), 1)
    liota = lax.broadcasted_iota(jnp.int32, (B, _LANES), 1)
    neg1 = jnp.full((B, _LANES), -1.0, jnp.float32)
    zero = jnp.zeros((B, _LANES), jnp.float32)
    ecnt = jnp.zeros((B, 1), jnp.int32)

    def cond(state):
        rowmax, ecnt = state[0], state[1]
        m = jnp.max(rowmax, axis=1, keepdims=True)
        active = (ecnt < _MAX_DET) & (m > _NEG_INF)
        return jnp.sum(active.astype(jnp.int32)) > 0

    def body(state):
        rowmax, ecnt, so, co, kx1, ky1, kx2, ky2, kar = state
        m = jnp.max(rowmax, axis=1, keepdims=True)
        active = (ecnt < _MAX_DET) & (m > _NEG_INF)
        rpick = jnp.min(jnp.where(rowmax == m, riota, R),
                        axis=1, keepdims=True)

        wrows, arows, prows = [], [], []
        for b in range(B):
            r_b = rpick[b, 0]
            wrows.append(w_ref[b, pl.ds(r_b, 1), :])
            arows.append(a_ref[b, pl.ds(r_b, 1), :])
            prows.append(p_ref[b, pl.ds(r_b, 1), :])
        wrow = jnp.concatenate(wrows, axis=0)
        arow = jnp.concatenate(arows, axis=0)
        prow = jnp.concatenate(prows, axis=0)

        oh = (wrow == m) & (liota == jnp.min(
            jnp.where(wrow == m, liota, _LANES), axis=1, keepdims=True))
        av = jnp.sum(jnp.where(oh, arow, 0), axis=1, keepdims=True)
        pv = jnp.sum(jnp.where(oh, prow, 0), axis=1, keepdims=True)
        px1 = (av & 2047).astype(jnp.float32)
        px2 = (((av >> 11) & 4095) - 2048).astype(jnp.float32)
        pcls = (av >> 23).astype(jnp.float32)
        py1 = (pv & 2047).astype(jnp.float32)
        py2 = (((pv >> 11) & 4095) - 2048).astype(jnp.float32)
        pa = (px2 - px1) * (py2 - py1)

        xx1 = jnp.maximum(px1, kx1)
        yy1 = jnp.maximum(py1, ky1)
        xx2 = jnp.minimum(px2, kx2)
        yy2 = jnp.minimum(py2, ky2)
        iw = jnp.maximum(xx2 - xx1, 0.0)
        ih = jnp.maximum(yy2 - yy1, 0.0)
        inter = iw * ih
        union = pa + kar - inter
        iou = jnp.where(union > 0, inter / jnp.where(union > 0, union, 1.0), 0.0)
        supv = (iou >= _NMS_THR) & (co == pcls)
        sup = jnp.max(supv.astype(jnp.int32), axis=1, keepdims=True) > 0
        emit = active & (~sup)

        pos = (liota == ecnt) & emit
        so = jnp.where(pos, m, so)
        co = jnp.where(pos, pcls, co)
        kx1 = jnp.where(pos, px1, kx1)
        ky1 = jnp.where(pos, py1, ky1)
        kx2 = jnp.where(pos, px2, kx2)
        ky2 = jnp.where(pos, py2, ky2)
        kar = jnp.where(pos, pa, kar)
        ecnt = ecnt + jnp.where(emit, 1, 0)

        wrow_new = jnp.where(oh & active, _NEG_INF, wrow)
        new_rm = jnp.max(wrow_new, axis=1, keepdims=True)
        for b in range(B):
            r_b = rpick[b, 0]
            w_ref[b, pl.ds(r_b, 1), :] = wrow_new[b:b + 1]
        rowmax = jnp.where((riota == rpick) & active, new_rm, rowmax)
        return rowmax, ecnt, so, co, kx1, ky1, kx2, ky2, kar

    state = (rowmax, ecnt, neg1, neg1, neg1, neg1, neg1, neg1, zero)
    state = lax.while_loop(cond, body, state)
    _, _, so, co, kx1, ky1, kx2, ky2, _ = state
    s_out_ref[...] = so
    c_out_ref[...] = co
    b_out_ref[:, 0, :] = kx1
    b_out_ref[:, 1, :] = ky1
    b_out_ref[:, 2, :] = kx2
    b_out_ref[:, 3, :] = ky2


def _run_decoder(cls4, reg4, anc4):
    B, _, R, L = cls4.shape
    return pl.pallas_call(
        _decoder_body,
        out_shape=[
            jax.ShapeDtypeStruct((B, _LANES), jnp.float32),
            jax.ShapeDtypeStruct((B, _LANES), jnp.float32),
            jax.ShapeDtypeStruct((B, 4, _LANES), jnp.float32),
        ],
        scratch_shapes=[
            pltpu.VMEM((B, R, L), jnp.float32),
            pltpu.VMEM((B, R, L), jnp.int32),
            pltpu.VMEM((B, R, L), jnp.int32),
        ],
    )(cls4, reg4, anc4)


def kernel(cls_heads, reg_heads, batch_anchors):
    cls = jnp.concatenate([cls_heads[i] for i in range(cls_heads.shape[0])], axis=1)
    reg = jnp.concatenate([reg_heads[i] for i in range(reg_heads.shape[0])], axis=1)
    anc = jnp.concatenate([batch_anchors[i] for i in range(batch_anchors.shape[0])], axis=1)
    B, N, C = cls.shape
    NP = -(-N // _LANES) * _LANES
    R = NP // _LANES
    clsT = jnp.pad(jnp.transpose(cls, (0, 2, 1)),
                   ((0, 0), (0, 0), (0, NP - N)), constant_values=-1.0)
    regT = jnp.pad(jnp.transpose(reg, (0, 2, 1)), ((0, 0), (0, 0), (0, NP - N)))
    ancT = jnp.pad(jnp.transpose(anc, (0, 2, 1)), ((0, 0), (0, 0), (0, NP - N)))
    so, co, bo = _run_decoder(clsT.reshape(B, C, R, _LANES),
                              regT.reshape(B, 4, R, _LANES),
                              ancT.reshape(B, 4, R, _LANES))
    s = so[:, :_MAX_DET]
    c = co[:, :_MAX_DET]
    b = jnp.transpose(bo, (0, 2, 1))[:, :_MAX_DET, :]
    return s, c, b


# speedup vs baseline: 1.0019x; 1.0019x over previous
import jax
import jax.numpy as jnp
from jax import lax
from jax.experimental import pallas as pl
from jax.experimental.pallas import tpu as pltpu

_IMAGE_W = 1024
_IMAGE_H = 1024
_MIN_SCORE = 0.05
_NMS_THR = 0.5
_MAX_DET = 100
_LANES = 128
_NEG_INF = float("-inf")


def _decoder_body(cls_ref, reg_ref, anc_ref,
                  s_out_ref, c_out_ref, b_out_ref,
                  w_ref, a_ref, p_ref):
    B, C, R, L = cls_ref.shape

    def class_step(c, carry):
        m, idx = carry
        v = cls_ref[:, c]
        better = v > m
        return jnp.where(better, v, m), jnp.where(better, c, idx)

    m0 = cls_ref[:, 0]
    idx0 = jnp.zeros((B, R, L), jnp.int32)
    scores, classes = lax.fori_loop(1, C, class_step, (m0, idx0))

    reg = reg_ref[...]
    anc = anc_ref[...]
    ax1, ay1, ax2, ay2 = anc[:, 0], anc[:, 1], anc[:, 2], anc[:, 3]
    aw = ax2 - ax1
    ah = ay2 - ay1
    acx = ax1 + 0.5 * aw
    acy = ay1 + 0.5 * ah
    tx = reg[:, 0] * 0.1
    ty = reg[:, 1] * 0.1
    tw = reg[:, 2] * 0.2
    th = reg[:, 3] * 0.2
    w = jnp.exp(tw) * aw
    h = jnp.exp(th) * ah
    cx = tx * aw + acx
    cy = ty * ah + acy
    xi1 = jnp.maximum((cx - 0.5 * w).astype(jnp.int32), 0)
    yi1 = jnp.maximum((cy - 0.5 * h).astype(jnp.int32), 0)
    xi2 = jnp.minimum((cx + 0.5 * w).astype(jnp.int32), _IMAGE_W - 1)
    yi2 = jnp.minimum((cy + 0.5 * h).astype(jnp.int32), _IMAGE_H - 1)

    wv = jnp.where(scores > _MIN_SCORE, scores, _NEG_INF)
    av_all = (jnp.clip(xi1, 0, 2047)
              | ((jnp.clip(xi2, -2048, 2047) + 2048) << 11)
              | (classes << 23))
    pv_all = (jnp.clip(yi1, 0, 2047)
              | ((jnp.clip(yi2, -2048, 2047) + 2048) << 11))
    for b in range(B):
        for r in range(R):
            w_ref[b * R + r] = wv[b, r][None]
            a_ref[b * R + r] = av_all[b, r][None]
            p_ref[b * R + r] = pv_all[b, r][None]

    rowmax = jnp.max(wv, axis=2)
    riota = lax.broadcasted_iota(jnp.int32, (B, R), 1)
    liota = lax.broadcasted_iota(jnp.int32, (B, _LANES), 1)
    neg1 = jnp.full((B, _LANES), -1.0, jnp.float32)
    zero = jnp.zeros((B, _LANES), jnp.float32)
    ecnt = jnp.zeros((B, 1), jnp.int32)

    def cond(state):
        rowmax, ecnt = state[0], state[1]
        m = jnp.max(rowmax, axis=1, keepdims=True)
        active = (ecnt < _MAX_DET) & (m > _NEG_INF)
        return jnp.sum(active.astype(jnp.int32)) > 0

    def body(state):
        rowmax, ecnt, so, co, kx1, ky1, kx2, ky2, kar = state
        m = jnp.max(rowmax, axis=1, keepdims=True)
        active = (ecnt < _MAX_DET) & (m > _NEG_INF)
        rpick = jnp.min(jnp.where(rowmax == m, riota, R),
                        axis=1, keepdims=True)

        wrows, arows, prows = [], [], []
        for b in range(B):
            g_b = b * R + rpick[b, 0]
            wrows.append(w_ref[pl.ds(g_b, 1), 0, :])
            arows.append(a_ref[pl.ds(g_b, 1), 0, :])
            prows.append(p_ref[pl.ds(g_b, 1), 0, :])
        wrow = jnp.concatenate(wrows, axis=0)
        arow = jnp.concatenate(arows, axis=0)
        prow = jnp.concatenate(prows, axis=0)

        oh = (wrow == m) & (liota == jnp.min(
            jnp.where(wrow == m, liota, _LANES), axis=1, keepdims=True))
        av = jnp.sum(jnp.where(oh, arow, 0), axis=1, keepdims=True)
        pv = jnp.sum(jnp.where(oh, prow, 0), axis=1, keepdims=True)
        px1 = (av & 2047).astype(jnp.float32)
        px2 = (((av >> 11) & 4095) - 2048).astype(jnp.float32)
        pcls = (av >> 23).astype(jnp.float32)
        py1 = (pv & 2047).astype(jnp.float32)
        py2 = (((pv >> 11) & 4095) - 2048).astype(jnp.float32)
        pa = (px2 - px1) * (py2 - py1)

        xx1 = jnp.maximum(px1, kx1)
        yy1 = jnp.maximum(py1, ky1)
        xx2 = jnp.minimum(px2, kx2)
        yy2 = jnp.minimum(py2, ky2)
        iw = jnp.maximum(xx2 - xx1, 0.0)
        ih = jnp.maximum(yy2 - yy1, 0.0)
        inter = iw * ih
        union = pa + kar - inter
        iou = jnp.where(union > 0, inter / jnp.where(union > 0, union, 1.0), 0.0)
        supv = (iou >= _NMS_THR) & (co == pcls)
        sup = jnp.max(supv.astype(jnp.int32), axis=1, keepdims=True) > 0
        emit = active & (~sup)

        pos = (liota == ecnt) & emit
        so = jnp.where(pos, m, so)
        co = jnp.where(pos, pcls, co)
        kx1 = jnp.where(pos, px1, kx1)
        ky1 = jnp.where(pos, py1, ky1)
        kx2 = jnp.where(pos, px2, kx2)
        ky2 = jnp.where(pos, py2, ky2)
        kar = jnp.where(pos, pa, kar)
        ecnt = ecnt + jnp.where(emit, 1, 0)

        wrow_new = jnp.where(oh & active, _NEG_INF, wrow)
        new_rm = jnp.max(wrow_new, axis=1, keepdims=True)
        for b in range(B):
            g_b = b * R + rpick[b, 0]
            w_ref[pl.ds(g_b, 1), 0, :] = wrow_new[b:b + 1]
        rowmax = jnp.where((riota == rpick) & active, new_rm, rowmax)
        return rowmax, ecnt, so, co, kx1, ky1, kx2, ky2, kar

    state = (rowmax, ecnt, neg1, neg1, neg1, neg1, neg1, neg1, zero)
    state = lax.while_loop(cond, body, state)
    _, _, so, co, kx1, ky1, kx2, ky2, _ = state
    s_out_ref[...] = so
    c_out_ref[...] = co
    b_out_ref[:, 0, :] = kx1
    b_out_ref[:, 1, :] = ky1
    b_out_ref[:, 2, :] = kx2
    b_out_ref[:, 3, :] = ky2


def _run_decoder(cls4, reg4, anc4):
    B, _, R, L = cls4.shape
    return pl.pallas_call(
        _decoder_body,
        out_shape=[
            jax.ShapeDtypeStruct((B, _LANES), jnp.float32),
            jax.ShapeDtypeStruct((B, _LANES), jnp.float32),
            jax.ShapeDtypeStruct((B, 4, _LANES), jnp.float32),
        ],
        scratch_shapes=[
            pltpu.VMEM((B * R, 1, L), jnp.float32),
            pltpu.VMEM((B * R, 1, L), jnp.int32),
            pltpu.VMEM((B * R, 1, L), jnp.int32),
        ],
    )(cls4, reg4, anc4)


def kernel(cls_heads, reg_heads, batch_anchors):
    cls = jnp.concatenate([cls_heads[i] for i in range(cls_heads.shape[0])], axis=1)
    reg = jnp.concatenate([reg_heads[i] for i in range(reg_heads.shape[0])], axis=1)
    anc = jnp.concatenate([batch_anchors[i] for i in range(batch_anchors.shape[0])], axis=1)
    B, N, C = cls.shape
    NP = -(-N // _LANES) * _LANES
    R = NP // _LANES
    clsT = jnp.pad(jnp.transpose(cls, (0, 2, 1)),
                   ((0, 0), (0, 0), (0, NP - N)), constant_values=-1.0)
    regT = jnp.pad(jnp.transpose(reg, (0, 2, 1)), ((0, 0), (0, 0), (0, NP - N)))
    ancT = jnp.pad(jnp.transpose(anc, (0, 2, 1)), ((0, 0), (0, 0), (0, NP - N)))
    so, co, bo = _run_decoder(clsT.reshape(B, C, R, _LANES),
                              regT.reshape(B, 4, R, _LANES),
                              ancT.reshape(B, 4, R, _LANES))
    s = so[:, :_MAX_DET]
    c = co[:, :_MAX_DET]
    b = jnp.transpose(bo, (0, 2, 1))[:, :_MAX_DET, :]
    return s, c, b
